# baseline (speedup 1.0000x reference)
import functools

import jax
import jax.numpy as jnp
from jax import lax
from jax.experimental import pallas as pl
from jax.experimental.pallas import tpu as pltpu
from jax.experimental.pallas import tpu_sc as plsc

N_FIELDS = 26
VOCAB = 100000
EMBED = 32
BATCH = 16384

NUM_CORES = 2
NUM_SUBCORES = 16
NUM_WORKERS = NUM_CORES * NUM_SUBCORES
BPW = BATCH // NUM_WORKERS


def _body(*refs):
    atb = refs[:N_FIELDS]
    tables = refs[N_FIELDS:2 * N_FIELDS]
    out = refs[2 * N_FIELDS]
    idx_v, rows_v, gat_sem, out_sem = refs[2 * N_FIELDS + 1:]

    wid = lax.axis_index("s") * NUM_CORES + lax.axis_index("c")
    base = wid * BPW

    for i in range(N_FIELDS):
        buf = i % 2
        if i >= 2:
            pltpu.make_async_copy(rows_v.at[buf],
                                  out.at[pl.ds(base, BPW), i - 2],
                                  out_sem.at[buf]).wait()
        pltpu.sync_copy(atb[i].at[pl.ds(base, BPW)], idx_v.at[buf])
        pltpu.async_copy(tables[i].at[idx_v.at[buf]], rows_v.at[buf],
                         gat_sem.at[buf]).wait()
        pltpu.make_async_copy(rows_v.at[buf], out.at[pl.ds(base, BPW), i],
                              out_sem.at[buf]).start()
    for i in (N_FIELDS - 2, N_FIELDS - 1):
        pltpu.make_async_copy(rows_v.at[i % 2], out.at[pl.ds(base, BPW), i],
                              out_sem.at[i % 2]).wait()


_sc_gather = pl.kernel(
    _body,
    out_type=jax.ShapeDtypeStruct((BATCH, N_FIELDS, EMBED), jnp.float32),
    mesh=plsc.VectorSubcoreMesh(core_axis_name="c", subcore_axis_name="s",
                                num_cores=NUM_CORES,
                                num_subcores=NUM_SUBCORES),
    scratch_types=[
        pltpu.VMEM((2, BPW), jnp.int32),
        pltpu.VMEM((2, BPW, EMBED), jnp.float32),
        pltpu.SemaphoreType.DMA((2,)),
        pltpu.SemaphoreType.DMA((2,)),
    ],
    compiler_params=pltpu.CompilerParams(use_tc_tiling_on_sc=False),
)


def kernel(atb_0, atb_1, atb_2, atb_3, atb_4, atb_5, atb_6, atb_7, atb_8,
           atb_9, atb_10, atb_11, atb_12, atb_13, atb_14, atb_15, atb_16,
           atb_17, atb_18, atb_19, atb_20, atb_21, atb_22, atb_23, atb_24,
           atb_25, W_0, W_1, W_2, W_3, W_4, W_5, W_6, W_7, W_8, W_9, W_10,
           W_11, W_12, W_13, W_14, W_15, W_16, W_17, W_18, W_19, W_20, W_21,
           W_22, W_23, W_24, W_25):
    atbs = [atb_0, atb_1, atb_2, atb_3, atb_4, atb_5, atb_6, atb_7, atb_8,
            atb_9, atb_10, atb_11, atb_12, atb_13, atb_14, atb_15, atb_16,
            atb_17, atb_18, atb_19, atb_20, atb_21, atb_22, atb_23, atb_24,
            atb_25]
    tables = [W_0, W_1, W_2, W_3, W_4, W_5, W_6, W_7, W_8, W_9, W_10, W_11,
              W_12, W_13, W_14, W_15, W_16, W_17, W_18, W_19, W_20, W_21,
              W_22, W_23, W_24, W_25]
    atbs = [a.astype(jnp.int32) for a in atbs]
    out = _sc_gather(*atbs, *tables)
    return out.reshape(BATCH, N_FIELDS * EMBED)


# speedup vs baseline: 3.9868x; 3.9868x over previous
import functools

import jax
import jax.numpy as jnp
from jax import lax
from jax.experimental import pallas as pl
from jax.experimental.pallas import tpu as pltpu
from jax.experimental.pallas import tpu_sc as plsc

N_FIELDS = 26
VOCAB = 100000
EMBED = 32
BATCH = 16384

NUM_CORES = 2
NUM_SUBCORES = 16
NUM_WORKERS = NUM_CORES * NUM_SUBCORES

CHUNK = 8192
UNROLL = 8


def _body(*refs):
    atb = refs[:N_FIELDS]
    tables_t = refs[N_FIELDS:2 * N_FIELDS]
    out_t = refs[2 * N_FIELDS]
    col_v, idx_v, gat_v = refs[2 * N_FIELDS + 1:]

    j = lax.axis_index("s") * NUM_CORES + lax.axis_index("c")

    for i in range(N_FIELDS):
        pltpu.sync_copy(tables_t[i].at[j], col_v)

        def chunk_step(c, _, atb_i=atb[i], row=i * EMBED + j):
            b0 = c * CHUNK
            pltpu.sync_copy(atb_i.at[pl.ds(b0, CHUNK)], idx_v)

            def gat_step(k, _):
                for u in range(UNROLL):
                    off = k * (16 * UNROLL) + u * 16
                    iv = idx_v[pl.ds(off, 16)]
                    gat_v[pl.ds(off, 16)] = plsc.load_gather(col_v, [iv])
                return _

            lax.fori_loop(0, CHUNK // (16 * UNROLL), gat_step, 0,
                          unroll=False)
            pltpu.sync_copy(gat_v, out_t.at[row, pl.ds(b0, CHUNK)])
            return _

        lax.fori_loop(0, BATCH // CHUNK, chunk_step, 0, unroll=False)


_sc_gather = pl.kernel(
    _body,
    out_type=jax.ShapeDtypeStruct((N_FIELDS * EMBED, BATCH), jnp.float32),
    mesh=plsc.VectorSubcoreMesh(core_axis_name="c", subcore_axis_name="s",
                                num_cores=NUM_CORES,
                                num_subcores=NUM_SUBCORES),
    scratch_types=[
        pltpu.VMEM((VOCAB,), jnp.float32),
        pltpu.VMEM((CHUNK,), jnp.int32),
        pltpu.VMEM((CHUNK,), jnp.float32),
    ],
    compiler_params=pltpu.CompilerParams(needs_layout_passes=False),
)


def kernel(atb_0, atb_1, atb_2, atb_3, atb_4, atb_5, atb_6, atb_7, atb_8,
           atb_9, atb_10, atb_11, atb_12, atb_13, atb_14, atb_15, atb_16,
           atb_17, atb_18, atb_19, atb_20, atb_21, atb_22, atb_23, atb_24,
           atb_25, W_0, W_1, W_2, W_3, W_4, W_5, W_6, W_7, W_8, W_9, W_10,
           W_11, W_12, W_13, W_14, W_15, W_16, W_17, W_18, W_19, W_20, W_21,
           W_22, W_23, W_24, W_25):
    atbs = [atb_0, atb_1, atb_2, atb_3, atb_4, atb_5, atb_6, atb_7, atb_8,
            atb_9, atb_10, atb_11, atb_12, atb_13, atb_14, atb_15, atb_16,
            atb_17, atb_18, atb_19, atb_20, atb_21, atb_22, atb_23, atb_24,
            atb_25]
    tables = [W_0, W_1, W_2, W_3, W_4, W_5, W_6, W_7, W_8, W_9, W_10, W_11,
              W_12, W_13, W_14, W_15, W_16, W_17, W_18, W_19, W_20, W_21,
              W_22, W_23, W_24, W_25]
    atbs = [a.astype(jnp.int32) for a in atbs]
    tables_t = [w.T for w in tables]
    out_t = _sc_gather(*atbs, *tables_t)
    return out_t.T.reshape(BATCH, N_FIELDS * EMBED)
